# baseline (speedup 1.0000x reference)
import jax
import jax.numpy as jnp
from jax import lax
from jax.experimental import pallas as pl
from jax.experimental.pallas import tpu as pltpu

_MARGIN = 0.1
_ROWS = 8192
_E = 64


def _tc_scan_body(p_ref, rta_ref, out_ref):
    r = rta_ref[...].reshape(1, _E)

    def step(i, r):
        s = p_ref[pl.ds(i, 1), :]
        t = r + s
        thr = jnp.sum(t, axis=1, keepdims=True) * (1.0 / _E) + _MARGIN
        m = t > thr
        anym = jnp.any(m, axis=1, keepdims=True)
        b = jnp.where(anym, t - s, t)
        ms = jnp.where(m, 0.0, s)
        n = jnp.sum(ms, axis=1, keepdims=True)
        nv = jnp.where(n == 0.0, 1.0, n)
        ms = ms / nv
        out_ref[pl.ds(i, 1), :] = ms
        return b + ms

    lax.fori_loop(0, _ROWS, step, r, unroll=4)


def _capacity_scan_tc(p, rta):
    return pl.pallas_call(
        _tc_scan_body,
        in_specs=[
            pl.BlockSpec(memory_space=pltpu.VMEM),
            pl.BlockSpec(memory_space=pltpu.VMEM),
        ],
        out_specs=pl.BlockSpec(memory_space=pltpu.VMEM),
        out_shape=jax.ShapeDtypeStruct((_ROWS, _E), jnp.float32),
    )(p, rta.reshape(1, _E))


def kernel(x, W1, b1, W2, b2, running_total_assignment):
    h = jax.nn.relu(x @ W1.T + b1)
    logits = h @ W2.T + b2
    p = jax.nn.softmax(logits, axis=1)
    return _capacity_scan_tc(p, running_total_assignment)


# speedup vs baseline: 1.3502x; 1.3502x over previous
import jax
import jax.numpy as jnp
from jax import lax
from jax.experimental import pallas as pl
from jax.experimental.pallas import tpu as pltpu

_MARGIN = 0.1
_ROWS = 8192
_E = 64


def _tc_scan_body(p_ref, rta_ref, out_ref):
    r = rta_ref[...].reshape(1, _E)

    def step(i, r):
        s = p_ref[pl.ds(i, 1), :]
        t = r + s
        thr = jnp.sum(t) * (1.0 / _E) + _MARGIN
        m = t > thr
        anym = jnp.any(m)
        b = jnp.where(anym, t - s, t)
        ms = jnp.where(m, 0.0, s)
        n = jnp.sum(ms)
        nv = jnp.where(n == 0.0, 1.0, n)
        ms = ms / nv
        out_ref[pl.ds(i, 1), :] = ms
        return b + ms

    lax.fori_loop(0, _ROWS, step, r, unroll=2)


def _capacity_scan_tc(p, rta):
    return pl.pallas_call(
        _tc_scan_body,
        in_specs=[
            pl.BlockSpec(memory_space=pltpu.VMEM),
            pl.BlockSpec(memory_space=pltpu.VMEM),
        ],
        out_specs=pl.BlockSpec(memory_space=pltpu.VMEM),
        out_shape=jax.ShapeDtypeStruct((_ROWS, _E), jnp.float32),
    )(p, rta.reshape(1, _E))


def kernel(x, W1, b1, W2, b2, running_total_assignment):
    h = jax.nn.relu(x @ W1.T + b1)
    logits = h @ W2.T + b2
    p = jax.nn.softmax(logits, axis=1)
    return _capacity_scan_tc(p, running_total_assignment)
